# baseline (speedup 1.0000x reference)
import jax
import jax.numpy as jnp
from jax.experimental import pallas as pl
from jax.experimental.pallas import tpu as pltpu

_E = 64
_TOP_K = 8
_D = 1024
_I = 768
_NBUF = 4


def _moe_body(x_ref, gate_ref, w1_hbm, w2_hbm, w3_hbm,
              out_ref, logits_ref,
              w1_buf, w2_buf, w3_buf, combine_ref, sems):

    def start(e):
        slot = e % _NBUF
        pltpu.make_async_copy(w1_hbm.at[e], w1_buf.at[slot],
                              sems.at[0, slot]).start()
        pltpu.make_async_copy(w2_hbm.at[e], w2_buf.at[slot],
                              sems.at[1, slot]).start()
        pltpu.make_async_copy(w3_hbm.at[e], w3_buf.at[slot],
                              sems.at[2, slot]).start()

    for e in range(_NBUF):
        start(e)

    x = x_ref[...]
    xb = x.astype(jnp.bfloat16)

    logits = jax.lax.dot_general(
        x, gate_ref[...], (((1,), (1,)), ((), ())),
        preferred_element_type=jnp.float32)
    logits_ref[...] = logits
    m = jnp.max(logits, axis=1, keepdims=True)
    ex = jnp.exp(logits - m)
    probs = ex / jnp.sum(ex, axis=1, keepdims=True)
    col = jax.lax.broadcasted_iota(jnp.int32, probs.shape, 1)
    masked = probs
    comb = jnp.zeros_like(probs)
    for _ in range(_TOP_K):
        maxv = jnp.max(masked, axis=1, keepdims=True)
        idx = jnp.min(jnp.where(masked == maxv, col, _E), axis=1,
                      keepdims=True)
        onehot = col == idx
        comb = comb + jnp.where(onehot, maxv, 0.0)
        masked = jnp.where(onehot, -1.0, masked)
    comb = comb / jnp.sum(comb, axis=1, keepdims=True)
    combine_ref[...] = comb

    def loop(e, _):
        slot = e % _NBUF
        pltpu.make_async_copy(w1_hbm.at[e], w1_buf.at[slot],
                              sems.at[0, slot]).wait()
        pltpu.make_async_copy(w2_hbm.at[e], w2_buf.at[slot],
                              sems.at[1, slot]).wait()
        pltpu.make_async_copy(w3_hbm.at[e], w3_buf.at[slot],
                              sems.at[2, slot]).wait()
        w1 = w1_buf[slot]
        w3 = w3_buf[slot]
        w2 = w2_buf[slot]
        g = jax.lax.dot_general(x, w1, (((1,), (1,)), ((), ())),
                                preferred_element_type=jnp.float32)
        u = jax.lax.dot_general(x, w3, (((1,), (1,)), ((), ())),
                                preferred_element_type=jnp.float32)
        h = (g * jax.lax.logistic(g)) * u
        y = jax.lax.dot_general(h, w2,
                                (((1,), (1,)), ((), ())),
                                preferred_element_type=jnp.float32)

        @pl.when(e + _NBUF < _E)
        def _prefetch():
            start(e + _NBUF)

        c = jnp.sum(jnp.where(col == e, combine_ref[...], 0.0), axis=1,
                    keepdims=True)

        @pl.when(e == 0)
        def _first():
            out_ref[...] = c * y

        @pl.when(e > 0)
        def _acc():
            out_ref[...] += c * y
        return 0

    jax.lax.fori_loop(0, _E, loop, 0)


def kernel(hidden_states, gate_w, w1, w2, w3):
    b, s, d = hidden_states.shape
    x = hidden_states.reshape(-1, d)
    t = x.shape[0]
    out, logits = pl.pallas_call(
        _moe_body,
        in_specs=[
            pl.BlockSpec(memory_space=pltpu.VMEM),
            pl.BlockSpec(memory_space=pltpu.VMEM),
            pl.BlockSpec(memory_space=pl.ANY),
            pl.BlockSpec(memory_space=pl.ANY),
            pl.BlockSpec(memory_space=pl.ANY),
        ],
        out_specs=[
            pl.BlockSpec(memory_space=pltpu.VMEM),
            pl.BlockSpec(memory_space=pltpu.VMEM),
        ],
        out_shape=[
            jax.ShapeDtypeStruct((t, _D), jnp.float32),
            jax.ShapeDtypeStruct((t, _E), jnp.float32),
        ],
        scratch_shapes=[
            pltpu.VMEM((_NBUF, _I, _D), jnp.float32),
            pltpu.VMEM((_NBUF, _D, _I), jnp.float32),
            pltpu.VMEM((_NBUF, _I, _D), jnp.float32),
            pltpu.VMEM((t, _E), jnp.float32),
            pltpu.SemaphoreType.DMA((3, _NBUF)),
        ],
    )(x, gate_w, w1, w2, w3)
    return out.reshape(b, s, d), logits


# speedup vs baseline: 1.0706x; 1.0706x over previous
import jax
import jax.numpy as jnp
from jax.experimental import pallas as pl
from jax.experimental.pallas import tpu as pltpu

_E = 64
_TOP_K = 8
_D = 1024
_I = 768
_NBUF = 4


def _moe_body(x_ref, gate_ref, w1_hbm, w2_hbm, w3_hbm,
              out_ref, logits_ref,
              w1_buf, w2_buf, w3_buf, combine_ref, used_vmem,
              order_smem, used_smem, sems, smem_sem):

    def start(e, slot):
        pltpu.make_async_copy(w1_hbm.at[e], w1_buf.at[slot],
                              sems.at[0, slot]).start()
        pltpu.make_async_copy(w2_hbm.at[e], w2_buf.at[slot],
                              sems.at[1, slot]).start()
        pltpu.make_async_copy(w3_hbm.at[e], w3_buf.at[slot],
                              sems.at[2, slot]).start()

    for e in range(_NBUF):
        start(e, e)
        order_smem[e] = e

    x = x_ref[...]
    xb = x.astype(jnp.bfloat16)

    logits = jax.lax.dot_general(
        x, gate_ref[...], (((1,), (1,)), ((), ())),
        preferred_element_type=jnp.float32)
    logits_ref[...] = logits
    m = jnp.max(logits, axis=1, keepdims=True)
    ex = jnp.exp(logits - m)
    probs = ex / jnp.sum(ex, axis=1, keepdims=True)
    col = jax.lax.broadcasted_iota(jnp.int32, probs.shape, 1)
    masked = probs
    comb = jnp.zeros_like(probs)
    for _ in range(_TOP_K):
        maxv = jnp.max(masked, axis=1, keepdims=True)
        idx = jnp.min(jnp.where(masked == maxv, col, _E), axis=1,
                      keepdims=True)
        onehot = col == idx
        comb = comb + jnp.where(onehot, maxv, 0.0)
        masked = jnp.where(onehot, -1.0, masked)
    comb = comb / jnp.sum(comb, axis=1, keepdims=True)
    combine_ref[...] = comb

    used_vmem[...] = (jnp.max(comb, axis=0, keepdims=True) > 0.0
                      ).astype(jnp.int32)
    pltpu.make_async_copy(used_vmem, used_smem, smem_sem).start()
    pltpu.make_async_copy(used_vmem, used_smem, smem_sem).wait()

    def compact(e, ptr):
        flag = used_smem[0, e]

        @pl.when(flag != 0)
        def _take():
            order_smem[ptr] = e

        return ptr + flag

    count = jax.lax.fori_loop(_NBUF, _E, compact, _NBUF)

    def loop(i, _):
        slot = i % _NBUF
        e = order_smem[i]
        pltpu.make_async_copy(w1_hbm.at[e], w1_buf.at[slot],
                              sems.at[0, slot]).wait()
        pltpu.make_async_copy(w2_hbm.at[e], w2_buf.at[slot],
                              sems.at[1, slot]).wait()
        pltpu.make_async_copy(w3_hbm.at[e], w3_buf.at[slot],
                              sems.at[2, slot]).wait()
        w1 = w1_buf[slot].astype(jnp.bfloat16)
        w3 = w3_buf[slot].astype(jnp.bfloat16)
        w2 = w2_buf[slot].astype(jnp.bfloat16)
        g = jax.lax.dot_general(xb, w1, (((1,), (1,)), ((), ())),
                                preferred_element_type=jnp.float32)
        u = jax.lax.dot_general(xb, w3, (((1,), (1,)), ((), ())),
                                preferred_element_type=jnp.float32)
        h = (g * jax.lax.logistic(g)) * u
        y = jax.lax.dot_general(h.astype(jnp.bfloat16), w2,
                                (((1,), (1,)), ((), ())),
                                preferred_element_type=jnp.float32)

        @pl.when(i + _NBUF < count)
        def _prefetch():
            start(order_smem[i + _NBUF], slot)

        c = jnp.sum(jnp.where(col == e, combine_ref[...], 0.0), axis=1,
                    keepdims=True)

        @pl.when(i == 0)
        def _first():
            out_ref[...] = c * y

        @pl.when(i > 0)
        def _acc():
            out_ref[...] += c * y
        return 0

    jax.lax.fori_loop(0, count, loop, 0)


def kernel(hidden_states, gate_w, w1, w2, w3):
    b, s, d = hidden_states.shape
    x = hidden_states.reshape(-1, d)
    t = x.shape[0]
    out, logits = pl.pallas_call(
        _moe_body,
        in_specs=[
            pl.BlockSpec(memory_space=pltpu.VMEM),
            pl.BlockSpec(memory_space=pltpu.VMEM),
            pl.BlockSpec(memory_space=pl.ANY),
            pl.BlockSpec(memory_space=pl.ANY),
            pl.BlockSpec(memory_space=pl.ANY),
        ],
        out_specs=[
            pl.BlockSpec(memory_space=pltpu.VMEM),
            pl.BlockSpec(memory_space=pltpu.VMEM),
        ],
        out_shape=[
            jax.ShapeDtypeStruct((t, _D), jnp.float32),
            jax.ShapeDtypeStruct((t, _E), jnp.float32),
        ],
        scratch_shapes=[
            pltpu.VMEM((_NBUF, _I, _D), jnp.float32),
            pltpu.VMEM((_NBUF, _D, _I), jnp.float32),
            pltpu.VMEM((_NBUF, _I, _D), jnp.float32),
            pltpu.VMEM((t, _E), jnp.float32),
            pltpu.VMEM((1, _E), jnp.int32),
            pltpu.SMEM((_E,), jnp.int32),
            pltpu.SMEM((1, _E), jnp.int32),
            pltpu.SemaphoreType.DMA((3, _NBUF)),
            pltpu.SemaphoreType.DMA,
        ],
    )(x, gate_w, w1, w2, w3)
    return out.reshape(b, s, d), logits
